# baseline (speedup 1.0000x reference)
import functools

import jax
import jax.numpy as jnp
from jax import lax
from jax.experimental import pallas as pl
from jax.experimental.pallas import tpu as pltpu
from jax.experimental.pallas import tpu_sc as plsc

_NW = 32
_C = 16


def _sc_add(x, emb_table):
    B, T, D = x.shape
    wpb = _NW // B
    t_per_w = T // wpb
    n_chunks = t_per_w // _C
    mesh = plsc.VectorSubcoreMesh(core_axis_name="c", subcore_axis_name="s")

    @functools.partial(
        pl.kernel, mesh=mesh,
        out_type=jax.ShapeDtypeStruct((B, T, D), jnp.float32),
        scratch_types=[
            pltpu.VMEM((_C, D), jnp.float32),
            pltpu.VMEM((_C, D), jnp.float32),
            pltpu.VMEM((_C, D), jnp.float32),
            pltpu.VMEM((_C, D), jnp.float32),
            pltpu.SemaphoreType.DMA,
            pltpu.SemaphoreType.DMA,
            pltpu.SemaphoreType.DMA,
            pltpu.SemaphoreType.DMA,
        ],
    )
    def k(x_hbm, emb_hbm, out_hbm, x0, x1, e0, e1, si0, si1, so0, so1):
        cid = lax.axis_index("c")
        sid = lax.axis_index("s")
        w = sid * 2 + cid
        b = w // wpb
        t_base = (w % wpb) * t_per_w
        xb = (x0, x1)
        eb = (e0, e1)
        sin = (si0, si1)
        sout = (so0, so1)

        def in_copies(g, p):
            t0 = t_base + g * _C
            return (
                pltpu.make_async_copy(x_hbm.at[b, pl.ds(t0, _C), :], xb[p], sin[p]),
                pltpu.make_async_copy(emb_hbm.at[pl.ds(t0, _C), :], eb[p], sin[p]),
            )

        def out_copy(g, p):
            t0 = t_base + g * _C
            return pltpu.make_async_copy(xb[p], out_hbm.at[b, pl.ds(t0, _C), :], sout[p])

        def compute(p):
            xv, ev = xb[p], eb[p]

            def row(r, carry):
                for kk in range(D // 16):
                    sl = pl.ds(kk * 16, 16)
                    xv[r, sl] = xv[r, sl] + ev[r, sl]
                return carry

            lax.fori_loop(0, _C, row, 0)

        for cpy in in_copies(0, 0):
            cpy.start()

        def outer(o, carry):
            for p in (0, 1):
                g = 2 * o + p
                for cpy in in_copies(g, p):
                    cpy.wait()

                @pl.when(g + 1 < n_chunks)
                def _():
                    @pl.when(g >= 1)
                    def _():
                        out_copy(g - 1, 1 - p).wait()

                    for cpy in in_copies(g + 1, 1 - p):
                        cpy.start()

                compute(p)
                out_copy(g, p).start()
            return carry

        lax.fori_loop(0, n_chunks // 2, outer, 0)
        out_copy(n_chunks - 2, (n_chunks - 2) % 2).wait()
        out_copy(n_chunks - 1, (n_chunks - 1) % 2).wait()

    return k(x, emb_table)


def kernel(x, emb_table):
    return _sc_add(x, emb_table)
